# Initial kernel scaffold; baseline (speedup 1.0000x reference)
import jax
import jax.numpy as jnp
from jax.experimental import pallas as pl


def kernel(features, Wself0, Wneigh0, b0, Wself1, Wneigh1, b1, Wself2, Wneigh2, b2, src, dst):
    raise NotImplementedError("write your pallas kernel here")


# speedup vs baseline: 3.0689x; 3.0689x over previous
import functools

import jax
import jax.numpy as jnp
from jax import lax
from jax.experimental import pallas as pl
from jax.experimental.pallas import tpu as pltpu
from jax.experimental.pallas import tpu_sc as plsc

N = 10000
NP = 10240
E = 160000
D = 256
H = 128
NC = 2
NS = 16
EPT = E // NS
CH = 80
NCH = EPT // CH
RPT = NP // NS


def _sc_mesh():
    return plsc.VectorSubcoreMesh(core_axis_name="c", subcore_axis_name="s")


@functools.partial(
    pl.kernel,
    out_type=jax.ShapeDtypeStruct((NP,), jnp.float32),
    mesh=_sc_mesh(),
    scratch_types=[
        pltpu.VMEM((CH,), jnp.int32),
        pltpu.VMEM((CH,), jnp.float32),
        pltpu.VMEM((RPT,), jnp.float32),
        pltpu.VMEM_SHARED((NP,), jnp.float32),
    ],
)
def _deg_kernel(dst_hbm, out_hbm, didx, ones, zbuf, dacc):
    c = lax.axis_index("c")
    s = lax.axis_index("s")
    for i in range(CH // 16):
        ones[pl.ds(i * 16, 16)] = jnp.ones((16,), jnp.float32)
    for i in range(RPT // 16):
        zbuf[pl.ds(i * 16, 16)] = jnp.zeros((16,), jnp.float32)
    pltpu.sync_copy(zbuf, dacc.at[pl.ds(s * RPT, RPT)])
    plsc.subcore_barrier()

    def body(k, carry):
        base = s * EPT + k * CH
        pltpu.sync_copy(dst_hbm.at[pl.ds(base, CH)], didx)
        pltpu.sync_copy(ones, dacc.at[didx], add=True)
        return carry

    lax.fori_loop(0, NCH, body, 0)
    plsc.subcore_barrier()

    @pl.when(c == 0)
    def _():
        pltpu.sync_copy(dacc.at[pl.ds(s * RPT, RPT)], out_hbm.at[pl.ds(s * RPT, RPT)])


@functools.partial(
    pl.kernel,
    out_type=jax.ShapeDtypeStruct((NC, NP, H), jnp.float32),
    mesh=_sc_mesh(),
    scratch_types=[
        pltpu.VMEM((CH,), jnp.int32),
        pltpu.VMEM((CH,), jnp.int32),
        pltpu.VMEM((CH, H), jnp.float32),
        pltpu.VMEM_SHARED((NP, H), jnp.float32),
        pltpu.SemaphoreType.DMA,
    ],
)
def _agg_kernel(h2_hbm, src2_hbm, dst_hbm, zeros_hbm, out_hbm,
                sidx, didx, rows, accum, sem):
    c = lax.axis_index("c")
    s = lax.axis_index("s")
    pltpu.sync_copy(zeros_hbm.at[pl.ds(s * RPT, RPT)],
                    accum.at[pl.ds(s * RPT, RPT)])
    plsc.subcore_barrier()

    def body(k, carry):
        base = s * EPT + k * CH
        pltpu.sync_copy(src2_hbm.at[pl.ds(c * E + base, CH)], sidx)
        pltpu.sync_copy(dst_hbm.at[pl.ds(base, CH)], didx)
        pltpu.async_copy(h2_hbm.at[sidx], rows, sem).wait()
        pltpu.sync_copy(rows, accum.at[didx], add=True)
        return carry

    lax.fori_loop(0, NCH, body, 0)
    plsc.subcore_barrier()
    pltpu.sync_copy(accum.at[pl.ds(s * RPT, RPT)],
                    out_hbm.at[c, pl.ds(s * RPT, RPT)])


@functools.partial(
    pl.kernel,
    out_type=(jax.ShapeDtypeStruct((E, D), jnp.float32),
              jax.ShapeDtypeStruct((E, D), jnp.float32)),
    mesh=_sc_mesh(),
    scratch_types=[
        pltpu.VMEM((CH,), jnp.int32),
        pltpu.VMEM((CH, D), jnp.float32),
        pltpu.SemaphoreType.DMA,
    ],
)
def _pair_gather_kernel(hn_hbm, src_hbm, dst_hbm, out_s_hbm, out_d_hbm,
                        idx, rows, sem):
    c = lax.axis_index("c")
    s = lax.axis_index("s")

    def run(idx_hbm, out_hbm):
        def body(k, carry):
            base = s * EPT + k * CH
            pltpu.sync_copy(idx_hbm.at[pl.ds(base, CH)], idx)
            pltpu.async_copy(hn_hbm.at[idx], rows, sem).wait()
            pltpu.sync_copy(rows, out_hbm.at[pl.ds(base, CH)])
            return carry
        lax.fori_loop(0, NCH, body, 0)

    @pl.when(c == 0)
    def _():
        run(src_hbm, out_s_hbm)

    @pl.when(c == 1)
    def _():
        run(dst_hbm, out_d_hbm)


_R = 1024


def _combine_body(relu, h_ref, a_ref, d_ref, ws_ref, wn_ref, b_ref, o_ref):
    inv = 1.0 / jnp.maximum(d_ref[...], 1.0)
    m0 = a_ref[0] * inv
    m1 = a_ref[1] * inv
    acc = (jnp.dot(h_ref[0], ws_ref[0:H, :], preferred_element_type=jnp.float32)
           + jnp.dot(h_ref[1], ws_ref[H:D, :], preferred_element_type=jnp.float32)
           + jnp.dot(m0, wn_ref[0:H, :], preferred_element_type=jnp.float32)
           + jnp.dot(m1, wn_ref[H:D, :], preferred_element_type=jnp.float32))
    acc = acc + b_ref[...][None, :]
    if relu:
        o_ref[0] = jnp.maximum(acc, 0.0)
    else:
        o_ref[...] = acc


_combine_in_specs = [
    pl.BlockSpec((NC, _R, H), lambda i, j: (0, i, 0)),
    pl.BlockSpec((NC, _R, H), lambda i, j: (0, i, 0)),
    pl.BlockSpec((_R, H), lambda i, j: (i, 0)),
    pl.BlockSpec((D, H), lambda i, j: (0, j)),
    pl.BlockSpec((D, H), lambda i, j: (0, j)),
    pl.BlockSpec((H,), lambda i, j: (j,)),
]


def _combine_hidden(h, aggp, degb, ws, wn, b):
    return pl.pallas_call(
        functools.partial(_combine_body, True),
        grid=(NP // _R, D // H),
        in_specs=_combine_in_specs,
        out_specs=pl.BlockSpec((1, _R, H), lambda i, j: (j, i, 0)),
        out_shape=jax.ShapeDtypeStruct((NC, NP, H), jnp.float32),
    )(h, aggp, degb, ws, wn, b)


def _combine_final(h, aggp, degb, ws, wn, b):
    return pl.pallas_call(
        functools.partial(_combine_body, False),
        grid=(NP // _R, D // H),
        in_specs=_combine_in_specs,
        out_specs=pl.BlockSpec((_R, H), lambda i, j: (i, j)),
        out_shape=jax.ShapeDtypeStruct((NP, D), jnp.float32),
    )(h, aggp, degb, ws, wn, b)


def kernel(features, Wself0, Wneigh0, b0, Wself1, Wneigh1, b1,
           Wself2, Wneigh2, b2, src, dst):
    params = [(Wself0, Wneigh0, b0), (Wself1, Wneigh1, b1),
              (Wself2, Wneigh2, b2)]
    f = jnp.pad(features, ((0, NP - N), (0, 0)))
    h = jnp.transpose(f.reshape(NP, NC, H), (1, 0, 2))
    src2 = jnp.concatenate([src, src + NP])
    zeros_hbm = jnp.zeros((NP, H), jnp.float32)

    deg = _deg_kernel(dst)
    degb = jnp.broadcast_to(deg[:, None], (NP, H))

    hn = None
    for l, (ws, wn, b) in enumerate(params):
        h2 = h.reshape(NC * NP, H)
        aggp = _agg_kernel(h2, src2, dst, zeros_hbm)
        if l < 2:
            h = _combine_hidden(h, aggp, degb, ws, wn, b)
        else:
            hn = _combine_final(h, aggp, degb, ws, wn, b)

    src_feat, dst_feat = _pair_gather_kernel(hn, src, dst)
    return (src_feat, dst_feat)


# speedup vs baseline: 5.2432x; 1.7085x over previous
import functools

import jax
import jax.numpy as jnp
from jax import lax
from jax.experimental import pallas as pl
from jax.experimental.pallas import tpu as pltpu
from jax.experimental.pallas import tpu_sc as plsc

N = 10000
NP = 10240
E = 160000
D = 256
H = 128
NC = 2
NS = 16
EPT = E // NS
CH = 80
NCH = EPT // CH
CPT = NCH
RPT = NP // NS


def _sc_mesh():
    return plsc.VectorSubcoreMesh(core_axis_name="c", subcore_axis_name="s")


@functools.partial(
    pl.kernel,
    out_type=jax.ShapeDtypeStruct((NP,), jnp.float32),
    mesh=_sc_mesh(),
    scratch_types=[
        pltpu.VMEM((CPT, CH), jnp.int32),
        pltpu.VMEM((CH,), jnp.float32),
        pltpu.VMEM((RPT,), jnp.float32),
        pltpu.VMEM_SHARED((NP,), jnp.float32),
    ],
)
def _deg_kernel(dst3_hbm, out_hbm, didx, ones, zbuf, dacc):
    c = lax.axis_index("c")
    s = lax.axis_index("s")
    for i in range(CH // 16):
        ones[pl.ds(i * 16, 16)] = jnp.ones((16,), jnp.float32)
    for i in range(RPT // 16):
        zbuf[pl.ds(i * 16, 16)] = jnp.zeros((16,), jnp.float32)
    pltpu.sync_copy(zbuf, dacc.at[pl.ds(s * RPT, RPT)])
    pltpu.sync_copy(dst3_hbm.at[s], didx)
    plsc.subcore_barrier()

    def body(k, carry):
        pltpu.sync_copy(ones, dacc.at[didx.at[k]], add=True)
        return carry

    lax.fori_loop(0, NCH, body, 0)
    plsc.subcore_barrier()

    @pl.when(c == 0)
    def _():
        pltpu.sync_copy(dacc.at[pl.ds(s * RPT, RPT)], out_hbm.at[pl.ds(s * RPT, RPT)])


@functools.partial(
    pl.kernel,
    out_type=jax.ShapeDtypeStruct((NC, NP, H), jnp.float32),
    mesh=_sc_mesh(),
    scratch_types=[
        pltpu.VMEM((CPT, CH), jnp.int32),
        pltpu.VMEM((CH,), jnp.int32),
        pltpu.VMEM((CH,), jnp.int32),
        pltpu.VMEM((CH, H), jnp.float32),
        pltpu.VMEM((CH, H), jnp.float32),
        pltpu.VMEM_SHARED((NP, H), jnp.float32),
        pltpu.SemaphoreType.DMA,
        pltpu.SemaphoreType.DMA,
        pltpu.SemaphoreType.DMA,
        pltpu.SemaphoreType.DMA,
    ],
)
def _agg_kernel(h2_hbm, src4_hbm, dst3_hbm, zeros_hbm, out_hbm,
                sidx, dbuf0, dbuf1, rows0, rows1, accum,
                gsem0, gsem1, dsem0, dsem1):
    c = lax.axis_index("c")
    s = lax.axis_index("s")
    pltpu.sync_copy(zeros_hbm.at[pl.ds(s * RPT, RPT)],
                    accum.at[pl.ds(s * RPT, RPT)])
    pltpu.sync_copy(src4_hbm.at[c, s], sidx)
    plsc.subcore_barrier()

    def issue_g(k, buf, sem):
        pltpu.async_copy(h2_hbm.at[sidx.at[k]], buf, sem)

    def wait_g(buf, sem):
        pltpu.make_async_copy(h2_hbm.at[pl.ds(0, CH)], buf, sem).wait()

    def issue_d(k, buf, sem):
        pltpu.async_copy(dst3_hbm.at[s, k], buf, sem)

    def wait_d(buf, sem):
        pltpu.make_async_copy(dst3_hbm.at[0, 0], buf, sem).wait()

    issue_g(0, rows0, gsem0)
    issue_d(0, dbuf0, dsem0)

    def body(i, carry):
        k0 = 2 * i

        @pl.when(k0 + 1 < NCH)
        def _():
            issue_d(k0 + 1, dbuf1, dsem1)

        wait_g(rows0, gsem0)

        @pl.when(k0 + 1 < NCH)
        def _():
            issue_g(k0 + 1, rows1, gsem1)

        wait_d(dbuf0, dsem0)
        pltpu.sync_copy(rows0, accum.at[dbuf0], add=True)

        @pl.when(k0 + 1 < NCH)
        def _():
            @pl.when(k0 + 2 < NCH)
            def _():
                issue_d(k0 + 2, dbuf0, dsem0)

            wait_g(rows1, gsem1)

            @pl.when(k0 + 2 < NCH)
            def _():
                issue_g(k0 + 2, rows0, gsem0)

            wait_d(dbuf1, dsem1)
            pltpu.sync_copy(rows1, accum.at[dbuf1], add=True)

        return carry

    lax.fori_loop(0, (NCH + 1) // 2, body, 0)
    plsc.subcore_barrier()
    pltpu.sync_copy(accum.at[pl.ds(s * RPT, RPT)],
                    out_hbm.at[c, pl.ds(s * RPT, RPT)])


@functools.partial(
    pl.kernel,
    out_type=(jax.ShapeDtypeStruct((E, D), jnp.float32),
              jax.ShapeDtypeStruct((E, D), jnp.float32)),
    mesh=_sc_mesh(),
    scratch_types=[
        pltpu.VMEM((CPT, CH), jnp.int32),
        pltpu.VMEM((CH, D), jnp.float32),
        pltpu.VMEM((CH, D), jnp.float32),
        pltpu.SemaphoreType.DMA,
        pltpu.SemaphoreType.DMA,
    ],
)
def _pair_gather_kernel(hn_hbm, src3_hbm, dst3_hbm, out_s_hbm, out_d_hbm,
                        idx, rows0, rows1, sem0, sem1):
    c = lax.axis_index("c")
    s = lax.axis_index("s")

    def run(idx3_hbm, out_hbm):
        pltpu.sync_copy(idx3_hbm.at[s], idx)

        def issue(k, buf, sem):
            pltpu.async_copy(hn_hbm.at[idx.at[k]], buf, sem)

        def wait(buf, sem):
            pltpu.make_async_copy(hn_hbm.at[pl.ds(0, CH)], buf, sem).wait()

        def store(k, buf):
            pltpu.sync_copy(buf, out_hbm.at[pl.ds(s * EPT + k * CH, CH)])

        issue(0, rows0, sem0)

        def body(i, carry):
            k0 = 2 * i
            wait(rows0, sem0)

            @pl.when(k0 + 1 < NCH)
            def _():
                issue(k0 + 1, rows1, sem1)

            store(k0, rows0)

            @pl.when(k0 + 1 < NCH)
            def _():
                wait(rows1, sem1)

                @pl.when(k0 + 2 < NCH)
                def _():
                    issue(k0 + 2, rows0, sem0)

                store(k0 + 1, rows1)

            return carry

        lax.fori_loop(0, (NCH + 1) // 2, body, 0)

    @pl.when(c == 0)
    def _():
        run(src3_hbm, out_s_hbm)

    @pl.when(c == 1)
    def _():
        run(dst3_hbm, out_d_hbm)


_R = 1024


def _combine_body(relu, h_ref, a_ref, d_ref, ws_ref, wn_ref, b_ref, o_ref):
    inv = 1.0 / jnp.maximum(d_ref[...], 1.0)
    m0 = a_ref[0] * inv
    m1 = a_ref[1] * inv
    acc = (jnp.dot(h_ref[0], ws_ref[0:H, :], preferred_element_type=jnp.float32)
           + jnp.dot(h_ref[1], ws_ref[H:D, :], preferred_element_type=jnp.float32)
           + jnp.dot(m0, wn_ref[0:H, :], preferred_element_type=jnp.float32)
           + jnp.dot(m1, wn_ref[H:D, :], preferred_element_type=jnp.float32))
    acc = acc + b_ref[...][None, :]
    if relu:
        o_ref[0] = jnp.maximum(acc, 0.0)
    else:
        o_ref[...] = acc


_combine_in_specs = [
    pl.BlockSpec((NC, _R, H), lambda i, j: (0, i, 0)),
    pl.BlockSpec((NC, _R, H), lambda i, j: (0, i, 0)),
    pl.BlockSpec((_R, H), lambda i, j: (i, 0)),
    pl.BlockSpec((D, H), lambda i, j: (0, j)),
    pl.BlockSpec((D, H), lambda i, j: (0, j)),
    pl.BlockSpec((H,), lambda i, j: (j,)),
]


def _combine_hidden(h, aggp, degb, ws, wn, b):
    return pl.pallas_call(
        functools.partial(_combine_body, True),
        grid=(NP // _R, D // H),
        in_specs=_combine_in_specs,
        out_specs=pl.BlockSpec((1, _R, H), lambda i, j: (j, i, 0)),
        out_shape=jax.ShapeDtypeStruct((NC, NP, H), jnp.float32),
    )(h, aggp, degb, ws, wn, b)


def _combine_final(h, aggp, degb, ws, wn, b):
    return pl.pallas_call(
        functools.partial(_combine_body, False),
        grid=(NP // _R, D // H),
        in_specs=_combine_in_specs,
        out_specs=pl.BlockSpec((_R, H), lambda i, j: (i, j)),
        out_shape=jax.ShapeDtypeStruct((NP, D), jnp.float32),
    )(h, aggp, degb, ws, wn, b)


def kernel(features, Wself0, Wneigh0, b0, Wself1, Wneigh1, b1,
           Wself2, Wneigh2, b2, src, dst):
    params = [(Wself0, Wneigh0, b0), (Wself1, Wneigh1, b1),
              (Wself2, Wneigh2, b2)]
    f = jnp.pad(features, ((0, NP - N), (0, 0)))
    h = jnp.transpose(f.reshape(NP, NC, H), (1, 0, 2))
    src3 = src.reshape(NS, CPT, CH)
    dst3 = dst.reshape(NS, CPT, CH)
    src4 = jnp.stack([src3, src3 + NP])
    zeros_hbm = jnp.zeros((NP, H), jnp.float32)

    deg = _deg_kernel(dst3)
    degb = jnp.broadcast_to(deg[:, None], (NP, H))

    hn = None
    for l, (ws, wn, b) in enumerate(params):
        h2 = h.reshape(NC * NP, H)
        aggp = _agg_kernel(h2, src4, dst3, zeros_hbm)
        if l < 2:
            h = _combine_hidden(h, aggp, degb, ws, wn, b)
        else:
            hn = _combine_final(h, aggp, degb, ws, wn, b)

    src_feat, dst_feat = _pair_gather_kernel(hn, src3, dst3)
    return (src_feat, dst_feat)


# speedup vs baseline: 6.5334x; 1.2461x over previous
import functools

import jax
import jax.numpy as jnp
from jax import lax
from jax.experimental import pallas as pl
from jax.experimental.pallas import tpu as pltpu
from jax.experimental.pallas import tpu_sc as plsc

N = 10000
NP = 10240
E = 160000
D = 256
H = 128
NC = 2
NS = 16
CH = 128
NCH = 79
EP = NS * NCH * CH
EPT = E // NS
GF = EPT // CH
GT = EPT - GF * CH
RPT = NP // NS


def _sc_mesh():
    return plsc.VectorSubcoreMesh(core_axis_name="c", subcore_axis_name="s")


@functools.partial(
    pl.kernel,
    out_type=jax.ShapeDtypeStruct((NP,), jnp.float32),
    mesh=_sc_mesh(),
    scratch_types=[
        pltpu.VMEM((NCH, CH), jnp.int32),
        pltpu.VMEM((CH,), jnp.float32),
        pltpu.VMEM((RPT,), jnp.float32),
        pltpu.VMEM_SHARED((NP,), jnp.float32),
    ],
)
def _deg_kernel(dst3_hbm, out_hbm, didx, ones, zbuf, dacc):
    c = lax.axis_index("c")
    s = lax.axis_index("s")
    for i in range(CH // 16):
        ones[pl.ds(i * 16, 16)] = jnp.ones((16,), jnp.float32)
    for i in range(RPT // 16):
        zbuf[pl.ds(i * 16, 16)] = jnp.zeros((16,), jnp.float32)
    pltpu.sync_copy(zbuf, dacc.at[pl.ds(s * RPT, RPT)])
    pltpu.sync_copy(dst3_hbm.at[s], didx)
    plsc.subcore_barrier()

    def body(k, carry):
        pltpu.sync_copy(ones, dacc.at[didx.at[k]], add=True)
        return carry

    lax.fori_loop(0, NCH, body, 0)
    plsc.subcore_barrier()

    @pl.when(c == 0)
    def _():
        pltpu.sync_copy(dacc.at[pl.ds(s * RPT, RPT)], out_hbm.at[pl.ds(s * RPT, RPT)])


@functools.partial(
    pl.kernel,
    out_type=jax.ShapeDtypeStruct((NC, NP, H), jnp.float32),
    mesh=_sc_mesh(),
    scratch_types=[
        pltpu.VMEM((CH,), jnp.int32),
        pltpu.VMEM((CH,), jnp.int32),
        pltpu.VMEM((CH,), jnp.int32),
        pltpu.VMEM((CH,), jnp.int32),
        pltpu.VMEM((CH, H), jnp.float32),
        pltpu.VMEM((CH, H), jnp.float32),
        pltpu.VMEM_SHARED((NP, H), jnp.float32),
        pltpu.SemaphoreType.DMA,
        pltpu.SemaphoreType.DMA,
        pltpu.SemaphoreType.DMA,
        pltpu.SemaphoreType.DMA,
        pltpu.SemaphoreType.DMA,
        pltpu.SemaphoreType.DMA,
    ],
)
def _agg_kernel(h2_hbm, src4_hbm, dst3_hbm, zeros_hbm, out_hbm,
                sbuf0, sbuf1, dbuf0, dbuf1, rows0, rows1, accum,
                ssem0, ssem1, dsem0, dsem1, gsem0, gsem1):
    c = lax.axis_index("c")
    s = lax.axis_index("s")
    pltpu.sync_copy(zeros_hbm.at[pl.ds(s * RPT, RPT)],
                    accum.at[pl.ds(s * RPT, RPT)])
    plsc.subcore_barrier()

    def issue_s(k, buf, sem):
        pltpu.async_copy(src4_hbm.at[c, s, k], buf, sem)

    def wait_s(buf, sem):
        pltpu.make_async_copy(src4_hbm.at[0, 0, 0], buf, sem).wait()

    def issue_d(k, buf, sem):
        pltpu.async_copy(dst3_hbm.at[s, k], buf, sem)

    def wait_d(buf, sem):
        pltpu.make_async_copy(dst3_hbm.at[0, 0], buf, sem).wait()

    def issue_g(sb, buf, sem):
        pltpu.async_copy(h2_hbm.at[sb], buf, sem)

    def wait_g(buf, sem):
        pltpu.make_async_copy(h2_hbm.at[pl.ds(0, CH)], buf, sem).wait()

    issue_s(0, sbuf0, ssem0)
    issue_s(1, sbuf1, ssem1)
    issue_d(0, dbuf0, dsem0)
    issue_d(1, dbuf1, dsem1)
    wait_s(sbuf0, ssem0)
    issue_g(sbuf0, rows0, gsem0)

    def body(i, carry):
        k0 = 2 * i

        @pl.when(k0 + 1 < NCH)
        def _():
            wait_s(sbuf1, ssem1)
            issue_g(sbuf1, rows1, gsem1)

        wait_g(rows0, gsem0)

        @pl.when(k0 + 2 < NCH)
        def _():
            issue_s(k0 + 2, sbuf0, ssem0)

        wait_d(dbuf0, dsem0)
        pltpu.sync_copy(rows0, accum.at[dbuf0], add=True)

        @pl.when(k0 + 2 < NCH)
        def _():
            issue_d(k0 + 2, dbuf0, dsem0)

        @pl.when(k0 + 1 < NCH)
        def _():
            @pl.when(k0 + 2 < NCH)
            def _():
                wait_s(sbuf0, ssem0)
                issue_g(sbuf0, rows0, gsem0)

            wait_g(rows1, gsem1)

            @pl.when(k0 + 3 < NCH)
            def _():
                issue_s(k0 + 3, sbuf1, ssem1)

            wait_d(dbuf1, dsem1)
            pltpu.sync_copy(rows1, accum.at[dbuf1], add=True)

            @pl.when(k0 + 3 < NCH)
            def _():
                issue_d(k0 + 3, dbuf1, dsem1)

        return carry

    lax.fori_loop(0, (NCH + 1) // 2, body, 0)
    plsc.subcore_barrier()
    pltpu.sync_copy(accum.at[pl.ds(s * RPT, RPT)],
                    out_hbm.at[c, pl.ds(s * RPT, RPT)])


@functools.partial(
    pl.kernel,
    out_type=(jax.ShapeDtypeStruct((E, D), jnp.float32),
              jax.ShapeDtypeStruct((E, D), jnp.float32)),
    mesh=_sc_mesh(),
    scratch_types=[
        pltpu.VMEM((EPT,), jnp.int32),
        pltpu.VMEM((CH, D), jnp.float32),
        pltpu.VMEM((CH, D), jnp.float32),
        pltpu.SemaphoreType.DMA,
        pltpu.SemaphoreType.DMA,
    ],
)
def _pair_gather_kernel(hn_hbm, src_hbm, dst_hbm, out_s_hbm, out_d_hbm,
                        idx, rows0, rows1, sem0, sem1):
    c = lax.axis_index("c")
    s = lax.axis_index("s")

    def run(eidx_hbm, out_hbm):
        pltpu.sync_copy(eidx_hbm.at[pl.ds(s * EPT, EPT)], idx)

        def issue(k, buf, sem):
            pltpu.async_copy(hn_hbm.at[idx.at[pl.ds(k * CH, CH)]], buf, sem)

        def wait(buf, sem):
            pltpu.make_async_copy(hn_hbm.at[pl.ds(0, CH)], buf, sem).wait()

        def store(k, buf):
            pltpu.sync_copy(buf, out_hbm.at[pl.ds(s * EPT + k * CH, CH)])

        issue(0, rows0, sem0)

        def body(i, carry):
            k0 = 2 * i
            wait(rows0, sem0)
            issue(k0 + 1, rows1, sem1)
            store(k0, rows0)
            wait(rows1, sem1)

            @pl.when(k0 + 2 < GF)
            def _():
                issue(k0 + 2, rows0, sem0)

            store(k0 + 1, rows1)
            return carry

        lax.fori_loop(0, GF // 2, body, 0)
        tail = pltpu.async_copy(
            hn_hbm.at[idx.at[pl.ds(GF * CH, GT)]], rows0.at[pl.ds(0, GT)], sem0)
        tail.wait()
        pltpu.sync_copy(rows0.at[pl.ds(0, GT)],
                        out_hbm.at[pl.ds(s * EPT + GF * CH, GT)])

    @pl.when(c == 0)
    def _():
        run(src_hbm, out_s_hbm)

    @pl.when(c == 1)
    def _():
        run(dst_hbm, out_d_hbm)


_R = 1024


def _combine_body(relu, h_ref, a_ref, d_ref, ws_ref, wn_ref, b_ref, o_ref):
    inv = 1.0 / jnp.maximum(d_ref[...], 1.0)
    m0 = a_ref[0] * inv
    m1 = a_ref[1] * inv
    acc = (jnp.dot(h_ref[0], ws_ref[0:H, :], preferred_element_type=jnp.float32)
           + jnp.dot(h_ref[1], ws_ref[H:D, :], preferred_element_type=jnp.float32)
           + jnp.dot(m0, wn_ref[0:H, :], preferred_element_type=jnp.float32)
           + jnp.dot(m1, wn_ref[H:D, :], preferred_element_type=jnp.float32))
    acc = acc + b_ref[...][None, :]
    if relu:
        o_ref[0] = jnp.maximum(acc, 0.0)
    else:
        o_ref[...] = acc


_combine_in_specs = [
    pl.BlockSpec((NC, _R, H), lambda i, j: (0, i, 0)),
    pl.BlockSpec((NC, _R, H), lambda i, j: (0, i, 0)),
    pl.BlockSpec((_R, H), lambda i, j: (i, 0)),
    pl.BlockSpec((D, H), lambda i, j: (0, j)),
    pl.BlockSpec((D, H), lambda i, j: (0, j)),
    pl.BlockSpec((H,), lambda i, j: (j,)),
]


def _combine_hidden(h, aggp, degb, ws, wn, b):
    return pl.pallas_call(
        functools.partial(_combine_body, True),
        grid=(NP // _R, D // H),
        in_specs=_combine_in_specs,
        out_specs=pl.BlockSpec((1, _R, H), lambda i, j: (j, i, 0)),
        out_shape=jax.ShapeDtypeStruct((NC, NP, H), jnp.float32),
    )(h, aggp, degb, ws, wn, b)


def _combine_final(h, aggp, degb, ws, wn, b):
    return pl.pallas_call(
        functools.partial(_combine_body, False),
        grid=(NP // _R, D // H),
        in_specs=_combine_in_specs,
        out_specs=pl.BlockSpec((_R, H), lambda i, j: (i, j)),
        out_shape=jax.ShapeDtypeStruct((NP, D), jnp.float32),
    )(h, aggp, degb, ws, wn, b)


def kernel(features, Wself0, Wneigh0, b0, Wself1, Wneigh1, b1,
           Wself2, Wneigh2, b2, src, dst):
    params = [(Wself0, Wneigh0, b0), (Wself1, Wneigh1, b1),
              (Wself2, Wneigh2, b2)]
    f = jnp.pad(features, ((0, NP - N), (0, 0)))
    h = jnp.transpose(f.reshape(NP, NC, H), (1, 0, 2))

    pad = jnp.arange(E, EP, dtype=jnp.int32)
    src_p = jnp.concatenate([src, pad % N])
    dst_p = jnp.concatenate([dst, N + pad % (NP - N)])
    src3 = src_p.reshape(NS, NCH, CH)
    dst3 = dst_p.reshape(NS, NCH, CH)
    src4 = jnp.stack([src3, src3 + NP])
    zeros_hbm = jnp.zeros((NP, H), jnp.float32)

    deg = _deg_kernel(dst3)
    degb = jnp.broadcast_to(deg[:, None], (NP, H))

    hn = None
    for l, (ws, wn, b) in enumerate(params):
        h2 = h.reshape(NC * NP, H)
        aggp = _agg_kernel(h2, src4, dst3, zeros_hbm)
        if l < 2:
            h = _combine_hidden(h, aggp, degb, ws, wn, b)
        else:
            hn = _combine_final(h, aggp, degb, ws, wn, b)

    src_feat, dst_feat = _pair_gather_kernel(hn, src, dst)
    return (src_feat, dst_feat)


# speedup vs baseline: 6.6580x; 1.0191x over previous
import functools

import jax
import jax.numpy as jnp
from jax import lax
from jax.experimental import pallas as pl
from jax.experimental.pallas import tpu as pltpu
from jax.experimental.pallas import tpu_sc as plsc

N = 10000
NP = 10240
E = 160000
D = 256
H = 128
NC = 2
NS = 16
CH = 128
NCH = 79
EP = NS * NCH * CH
EPT = E // NS
GF = EPT // CH
GT = EPT - GF * CH
RPT = NP // NS


def _sc_mesh():
    return plsc.VectorSubcoreMesh(core_axis_name="c", subcore_axis_name="s")


@functools.partial(
    pl.kernel,
    out_type=jax.ShapeDtypeStruct((NP,), jnp.float32),
    mesh=_sc_mesh(),
    scratch_types=[
        pltpu.VMEM((NCH, CH), jnp.int32),
        pltpu.VMEM((CH,), jnp.float32),
        pltpu.VMEM((RPT,), jnp.float32),
        pltpu.VMEM_SHARED((NP,), jnp.float32),
    ],
)
def _deg_kernel(dst3_hbm, out_hbm, didx, ones, zbuf, dacc):
    c = lax.axis_index("c")
    s = lax.axis_index("s")
    for i in range(CH // 16):
        ones[pl.ds(i * 16, 16)] = jnp.ones((16,), jnp.float32)
    for i in range(RPT // 16):
        zbuf[pl.ds(i * 16, 16)] = jnp.zeros((16,), jnp.float32)
    pltpu.sync_copy(zbuf, dacc.at[pl.ds(s * RPT, RPT)])
    pltpu.sync_copy(dst3_hbm.at[s], didx)
    plsc.subcore_barrier()

    def body(k, carry):
        pltpu.sync_copy(ones, dacc.at[didx.at[k]], add=True)
        return carry

    lax.fori_loop(0, NCH, body, 0)
    plsc.subcore_barrier()

    @pl.when(c == 0)
    def _():
        pltpu.sync_copy(dacc.at[pl.ds(s * RPT, RPT)], out_hbm.at[pl.ds(s * RPT, RPT)])


@functools.partial(
    pl.kernel,
    out_type=jax.ShapeDtypeStruct((NC, NP, H), jnp.float32),
    mesh=_sc_mesh(),
    scratch_types=[
        pltpu.VMEM((CH,), jnp.int32),
        pltpu.VMEM((CH,), jnp.int32),
        pltpu.VMEM((CH,), jnp.int32),
        pltpu.VMEM((CH,), jnp.int32),
        pltpu.VMEM((CH, H), jnp.float32),
        pltpu.VMEM((CH, H), jnp.float32),
        pltpu.VMEM_SHARED((NP, H), jnp.float32),
        pltpu.SemaphoreType.DMA,
        pltpu.SemaphoreType.DMA,
        pltpu.SemaphoreType.DMA,
        pltpu.SemaphoreType.DMA,
        pltpu.SemaphoreType.DMA,
        pltpu.SemaphoreType.DMA,
    ],
)
def _agg_kernel(h2_hbm, src4_hbm, dst3_hbm, zeros_hbm, out_hbm,
                sbuf0, sbuf1, dbuf0, dbuf1, rows0, rows1, accum,
                ssem0, ssem1, dsem0, dsem1, gsem0, gsem1):
    c = lax.axis_index("c")
    s = lax.axis_index("s")
    pltpu.sync_copy(zeros_hbm.at[pl.ds(s * RPT, RPT)],
                    accum.at[pl.ds(s * RPT, RPT)])
    plsc.subcore_barrier()

    def issue_s(k, buf, sem):
        pltpu.async_copy(src4_hbm.at[c, s, k], buf, sem)

    def wait_s(buf, sem):
        pltpu.make_async_copy(src4_hbm.at[0, 0, 0], buf, sem).wait()

    def issue_d(k, buf, sem):
        pltpu.async_copy(dst3_hbm.at[s, k], buf, sem)

    def wait_d(buf, sem):
        pltpu.make_async_copy(dst3_hbm.at[0, 0], buf, sem).wait()

    def issue_g(sb, buf, sem):
        pltpu.async_copy(h2_hbm.at[sb], buf, sem)

    def wait_g(buf, sem):
        pltpu.make_async_copy(h2_hbm.at[pl.ds(0, CH)], buf, sem).wait()

    issue_s(0, sbuf0, ssem0)
    issue_s(1, sbuf1, ssem1)
    issue_d(0, dbuf0, dsem0)
    issue_d(1, dbuf1, dsem1)
    wait_s(sbuf0, ssem0)
    issue_g(sbuf0, rows0, gsem0)

    def body(i, carry):
        k0 = 2 * i

        @pl.when(k0 + 1 < NCH)
        def _():
            wait_s(sbuf1, ssem1)
            issue_g(sbuf1, rows1, gsem1)

        wait_g(rows0, gsem0)

        @pl.when(k0 + 2 < NCH)
        def _():
            issue_s(k0 + 2, sbuf0, ssem0)

        wait_d(dbuf0, dsem0)
        pltpu.sync_copy(rows0, accum.at[dbuf0], add=True)

        @pl.when(k0 + 2 < NCH)
        def _():
            issue_d(k0 + 2, dbuf0, dsem0)

        @pl.when(k0 + 1 < NCH)
        def _():
            @pl.when(k0 + 2 < NCH)
            def _():
                wait_s(sbuf0, ssem0)
                issue_g(sbuf0, rows0, gsem0)

            wait_g(rows1, gsem1)

            @pl.when(k0 + 3 < NCH)
            def _():
                issue_s(k0 + 3, sbuf1, ssem1)

            wait_d(dbuf1, dsem1)
            pltpu.sync_copy(rows1, accum.at[dbuf1], add=True)

            @pl.when(k0 + 3 < NCH)
            def _():
                issue_d(k0 + 3, dbuf1, dsem1)

        return carry

    lax.fori_loop(0, (NCH + 1) // 2, body, 0)
    plsc.subcore_barrier()
    pltpu.sync_copy(accum.at[pl.ds(s * RPT, RPT)],
                    out_hbm.at[c, pl.ds(s * RPT, RPT)])


@functools.partial(
    pl.kernel,
    out_type=(jax.ShapeDtypeStruct((E, D), jnp.float32),
              jax.ShapeDtypeStruct((E, D), jnp.float32)),
    mesh=_sc_mesh(),
    scratch_types=[
        pltpu.VMEM((EPT,), jnp.int32),
        pltpu.VMEM((CH, D), jnp.float32),
        pltpu.VMEM((CH, D), jnp.float32),
        pltpu.VMEM((CH, D), jnp.float32),
        pltpu.VMEM((GT, D), jnp.float32),
        pltpu.SemaphoreType.DMA,
        pltpu.SemaphoreType.DMA,
        pltpu.SemaphoreType.DMA,
        pltpu.SemaphoreType.DMA,
        pltpu.SemaphoreType.DMA,
        pltpu.SemaphoreType.DMA,
    ],
)
def _pair_gather_kernel(hn_hbm, src_hbm, dst_hbm, out_s_hbm, out_d_hbm,
                        idx, rows0, rows1, rows2, tbuf,
                        g0, g1, g2, s0, s1, s2):
    c = lax.axis_index("c")
    s = lax.axis_index("s")
    rows = [rows0, rows1, rows2]
    gsem = [g0, g1, g2]
    ssem = [s0, s1, s2]

    def run(eidx_hbm, out_hbm):
        pltpu.sync_copy(eidx_hbm.at[pl.ds(s * EPT, EPT)], idx)

        def issue(k, buf, sem):
            pltpu.async_copy(hn_hbm.at[idx.at[pl.ds(k * CH, CH)]], buf, sem)

        def wait_g(buf, sem):
            pltpu.make_async_copy(hn_hbm.at[pl.ds(0, CH)], buf, sem).wait()

        def store(k, buf, sem):
            pltpu.async_copy(buf, out_hbm.at[pl.ds(s * EPT + k * CH, CH)], sem)

        def wait_st(buf, sem):
            pltpu.make_async_copy(buf, out_hbm.at[pl.ds(0, CH)], sem).wait()

        issue(0, rows0, g0)
        issue(1, rows1, g1)

        def body(i, carry):
            k0 = 3 * i
            for o in range(3):
                k = k0 + o
                wait_g(rows[o], gsem[o])
                store(k, rows[o], ssem[o])
                nslot = (o + 2) % 3

                @pl.when(k + 2 < GF)
                def _(k=k, o=o, nslot=nslot):
                    @pl.when(k >= 1)
                    def _():
                        wait_st(rows[nslot], ssem[nslot])

                    issue(k + 2, rows[nslot], gsem[nslot])

            return carry

        lax.fori_loop(0, GF // 3, body, 0)
        for o in range(3):
            wait_st(rows[o], ssem[o])
        pltpu.async_copy(
            hn_hbm.at[idx.at[pl.ds(GF * CH, GT)]], tbuf, g0).wait()
        pltpu.sync_copy(tbuf, out_hbm.at[pl.ds(s * EPT + GF * CH, GT)])

    @pl.when(c == 0)
    def _():
        run(src_hbm, out_s_hbm)

    @pl.when(c == 1)
    def _():
        run(dst_hbm, out_d_hbm)


_R = 1024


def _self_body(h_ref, ws_ref, b_ref, o_ref):
    o_ref[0] = (
        jnp.dot(h_ref[0], ws_ref[0:H, :], preferred_element_type=jnp.float32)
        + jnp.dot(h_ref[1], ws_ref[H:D, :], preferred_element_type=jnp.float32)
        + b_ref[...][None, :])


def _self_part(h, ws, b):
    return pl.pallas_call(
        _self_body,
        grid=(NP // _R, D // H),
        in_specs=[
            pl.BlockSpec((NC, _R, H), lambda i, j: (0, i, 0)),
            pl.BlockSpec((D, H), lambda i, j: (0, j)),
            pl.BlockSpec((H,), lambda i, j: (j,)),
        ],
        out_specs=pl.BlockSpec((1, _R, H), lambda i, j: (j, i, 0)),
        out_shape=jax.ShapeDtypeStruct((NC, NP, H), jnp.float32),
    )(h, ws, b)


def _neigh_body(relu, s_ref, a_ref, d_ref, wn_ref, o_ref):
    inv = 1.0 / jnp.maximum(d_ref[...], 1.0)
    m0 = a_ref[0] * inv
    m1 = a_ref[1] * inv
    acc = (s_ref[0]
           + jnp.dot(m0, wn_ref[0:H, :], preferred_element_type=jnp.float32)
           + jnp.dot(m1, wn_ref[H:D, :], preferred_element_type=jnp.float32))
    if relu:
        o_ref[0] = jnp.maximum(acc, 0.0)
    else:
        o_ref[...] = acc


_neigh_in_specs = [
    pl.BlockSpec((1, _R, H), lambda i, j: (j, i, 0)),
    pl.BlockSpec((NC, _R, H), lambda i, j: (0, i, 0)),
    pl.BlockSpec((_R, H), lambda i, j: (i, 0)),
    pl.BlockSpec((D, H), lambda i, j: (0, j)),
]


def _neigh_hidden(sp, aggp, degb, wn):
    return pl.pallas_call(
        functools.partial(_neigh_body, True),
        grid=(NP // _R, D // H),
        in_specs=_neigh_in_specs,
        out_specs=pl.BlockSpec((1, _R, H), lambda i, j: (j, i, 0)),
        out_shape=jax.ShapeDtypeStruct((NC, NP, H), jnp.float32),
    )(sp, aggp, degb, wn)


def _neigh_final(sp, aggp, degb, wn):
    return pl.pallas_call(
        functools.partial(_neigh_body, False),
        grid=(NP // _R, D // H),
        in_specs=_neigh_in_specs,
        out_specs=pl.BlockSpec((_R, H), lambda i, j: (i, j)),
        out_shape=jax.ShapeDtypeStruct((NP, D), jnp.float32),
    )(sp, aggp, degb, wn)


def kernel(features, Wself0, Wneigh0, b0, Wself1, Wneigh1, b1,
           Wself2, Wneigh2, b2, src, dst):
    params = [(Wself0, Wneigh0, b0), (Wself1, Wneigh1, b1),
              (Wself2, Wneigh2, b2)]
    f = jnp.pad(features, ((0, NP - N), (0, 0)))
    h = jnp.transpose(f.reshape(NP, NC, H), (1, 0, 2))

    pad = jnp.arange(E, EP, dtype=jnp.int32)
    src_p = jnp.concatenate([src, pad % N])
    dst_p = jnp.concatenate([dst, N + pad % (NP - N)])
    src3 = src_p.reshape(NS, NCH, CH)
    dst3 = dst_p.reshape(NS, NCH, CH)
    src4 = jnp.stack([src3, src3 + NP])
    zeros_hbm = jnp.zeros((NP, H), jnp.float32)

    deg = _deg_kernel(dst3)
    degb = jnp.broadcast_to(deg[:, None], (NP, H))

    hn = None
    for l, (ws, wn, b) in enumerate(params):
        h2 = h.reshape(NC * NP, H)
        aggp = _agg_kernel(h2, src4, dst3, zeros_hbm)
        sp = _self_part(h, ws, b)
        if l < 2:
            h = _neigh_hidden(sp, aggp, degb, wn)
        else:
            hn = _neigh_final(sp, aggp, degb, wn)

    src_feat, dst_feat = _pair_gather_kernel(hn, src, dst)
    return (src_feat, dst_feat)
